# baseline (speedup 1.0000x reference)
import functools

import jax
import jax.numpy as jnp
from jax import lax
from jax.experimental import pallas as pl
from jax.experimental.pallas import tpu as pltpu
from jax.experimental.pallas import tpu_sc as plsc

_HIGH = jax.lax.Precision.HIGHEST


def _sc_gather(user, item, cat, user_table, item_table, cat_table):
    B = user.shape[0]
    info = plsc.get_sparse_core_info()
    nc, ns = info.num_cores, info.num_subcores
    nw = nc * ns
    bpw = B // nw
    eu = user_table.shape[1]
    ec = cat_table.shape[1]
    mesh = plsc.VectorSubcoreMesh(core_axis_name="c", subcore_axis_name="s")

    @functools.partial(
        pl.kernel,
        mesh=mesh,
        compiler_params=pltpu.CompilerParams(use_tc_tiling_on_sc=False),
        out_type=[
            jax.ShapeDtypeStruct((B, eu), jnp.float32),
            jax.ShapeDtypeStruct((B, eu), jnp.float32),
            jax.ShapeDtypeStruct((B, ec), jnp.float32),
        ],
        scratch_types=[
            pltpu.VMEM((bpw,), jnp.int32),
            pltpu.VMEM((bpw, eu), jnp.float32),
            pltpu.VMEM((bpw,), jnp.int32),
            pltpu.VMEM((bpw, eu), jnp.float32),
            pltpu.VMEM((bpw,), jnp.int32),
            pltpu.VMEM((bpw, ec), jnp.float32),
            pltpu.SemaphoreType.DMA,
        ],
    )
    def k(user_hbm, item_hbm, cat_hbm, ut_hbm, it_hbm, ct_hbm,
          u_out, i_out, c_out, uidx, urows, iidx, irows, cidx, crows, sem):
        wid = lax.axis_index("s") * nc + lax.axis_index("c")
        base = wid * bpw
        pltpu.sync_copy(user_hbm.at[pl.ds(base, bpw)], uidx)
        pltpu.sync_copy(item_hbm.at[pl.ds(base, bpw)], iidx)
        pltpu.sync_copy(cat_hbm.at[pl.ds(base, bpw)], cidx)
        cu = pltpu.async_copy(ut_hbm.at[uidx], urows, sem)
        ci = pltpu.async_copy(it_hbm.at[iidx], irows, sem)
        cc = pltpu.async_copy(ct_hbm.at[cidx], crows, sem)
        cu.wait()
        ci.wait()
        cc.wait()
        pltpu.sync_copy(urows, u_out.at[pl.ds(base, bpw)])
        pltpu.sync_copy(irows, i_out.at[pl.ds(base, bpw)])
        pltpu.sync_copy(crows, c_out.at[pl.ds(base, bpw)])

    return k(user, item, cat, user_table, item_table, cat_table)


_BLK = 2048


def _h_body(u_ref, i_ref, c_ref, d_ref, dwt_ref, db_ref,
            w1u_ref, w1i_ref, w1c_ref, w1d_ref, b1_ref,
            h_ref, sum_ref, sq_ref):
    dd = jnp.maximum(
        jnp.dot(d_ref[...], dwt_ref[...], precision=_HIGH) + db_ref[...], 0.0)
    h = (jnp.dot(u_ref[...], w1u_ref[...], precision=_HIGH)
         + jnp.dot(i_ref[...], w1i_ref[...], precision=_HIGH)
         + jnp.dot(c_ref[...], w1c_ref[...], precision=_HIGH)
         + jnp.dot(dd, w1d_ref[...], precision=_HIGH)
         + b1_ref[...])
    h_ref[...] = h
    sum_ref[...] = jnp.sum(h, axis=0, keepdims=True)[None]
    sq_ref[...] = jnp.sum(h * h, axis=0, keepdims=True)[None]


def _norm_body(h_ref, sum_ref, sq_ref, g_ref, bb_ref,
               w2t_ref, b2_ref, wot_ref, bo_ref, o_ref, *, batch):
    mean = jnp.sum(sum_ref[...], axis=0) / batch
    var = jnp.sum(sq_ref[...], axis=0) / batch - mean * mean
    h = h_ref[...]
    hn = (h - mean) * jax.lax.rsqrt(var + 1e-5) * g_ref[...] + bb_ref[...]
    x = jnp.maximum(hn, 0.0)
    x = jnp.maximum(
        jnp.dot(x, w2t_ref[...], precision=_HIGH) + b2_ref[...], 0.0)
    o_ref[...] = jnp.dot(x, wot_ref[...], precision=_HIGH) + bo_ref[...]


def _tc_mlp(u, i, c, dense, dense_W, dense_b, fc1_W, fc1_b,
            bn_gamma, bn_beta, fc2_W, fc2_b, out_W, out_b):
    B = u.shape[0]
    eu = u.shape[1]
    ec = c.shape[1]
    nb = B // _BLK
    w1t = fc1_W.T
    hdim = fc1_W.shape[0]

    def rows(bs):
        return pl.BlockSpec((_BLK, bs), lambda b: (b, 0))

    def full(shape):
        return pl.BlockSpec(shape, lambda b: (0,) * len(shape))

    h, sums, sqs = pl.pallas_call(
        _h_body,
        grid=(nb,),
        in_specs=[rows(eu), rows(eu), rows(ec), rows(2),
                  full((2, 8)), full((1, 8)),
                  full((eu, hdim)), full((eu, hdim)), full((ec, hdim)),
                  full((8, hdim)), full((1, hdim))],
        out_specs=[rows(hdim),
                   pl.BlockSpec((1, 1, hdim), lambda b: (b, 0, 0)),
                   pl.BlockSpec((1, 1, hdim), lambda b: (b, 0, 0))],
        out_shape=[jax.ShapeDtypeStruct((B, hdim), jnp.float32),
                   jax.ShapeDtypeStruct((nb, 1, hdim), jnp.float32),
                   jax.ShapeDtypeStruct((nb, 1, hdim), jnp.float32)],
    )(u, i, c, dense, dense_W.T, dense_b[None, :],
      w1t[:eu], w1t[eu:2 * eu], w1t[2 * eu:2 * eu + ec], w1t[2 * eu + ec:],
      fc1_b[None, :])

    return pl.pallas_call(
        functools.partial(_norm_body, batch=float(B)),
        grid=(nb,),
        in_specs=[rows(hdim), full((nb, 1, hdim)), full((nb, 1, hdim)),
                  full((1, hdim)), full((1, hdim)),
                  full((hdim, 32)), full((1, 32)), full((32, 1)),
                  full((1, 1))],
        out_specs=rows(1),
        out_shape=jax.ShapeDtypeStruct((B, 1), jnp.float32),
    )(h, sums, sqs, bn_gamma[None, :], bn_beta[None, :],
      fc2_W.T, fc2_b[None, :], out_W.T, out_b[None, :])


def kernel(user, item, cat, dense, user_table, item_table, cat_table,
           dense_W, dense_b, fc1_W, fc1_b, bn_gamma, bn_beta,
           fc2_W, fc2_b, out_W, out_b):
    u, i, c = _sc_gather(
        user.astype(jnp.int32), item.astype(jnp.int32), cat.astype(jnp.int32),
        user_table, item_table, cat_table)
    return _tc_mlp(u, i, c, dense, dense_W, dense_b, fc1_W, fc1_b,
                   bn_gamma, bn_beta, fc2_W, fc2_b, out_W, out_b)


# speedup vs baseline: 6.1473x; 6.1473x over previous
import functools

import jax
import jax.numpy as jnp
from jax import lax
from jax.experimental import pallas as pl
from jax.experimental.pallas import tpu as pltpu
from jax.experimental.pallas import tpu_sc as plsc

_HIGH = jax.lax.Precision.HIGHEST

_CHUNK = 128


def _sc_gather_cat(cat, cat_table):
    B = cat.shape[0]
    info = plsc.get_sparse_core_info()
    nc, ns = info.num_cores, info.num_subcores
    nw = nc * ns
    bpw = B // nw
    ec = cat_table.shape[1]
    ct_flat = cat_table.reshape(-1)
    mesh = plsc.VectorSubcoreMesh(core_axis_name="c", subcore_axis_name="s")
    nchunks = bpw // _CHUNK

    @functools.partial(
        pl.kernel,
        mesh=mesh,
        compiler_params=pltpu.CompilerParams(needs_layout_passes=False),
        out_type=jax.ShapeDtypeStruct((B, 128), jnp.float32),
        scratch_types=[
            pltpu.VMEM((bpw,), jnp.int32),
            pltpu.VMEM((ct_flat.shape[0],), jnp.float32),
            pltpu.VMEM((_CHUNK, 128), jnp.float32),
        ],
    )
    def k(cat_hbm, ct_hbm, c_out, cidx, ctab, csel):
        wid = lax.axis_index("s") * nc + lax.axis_index("c")
        base = wid * bpw
        pltpu.sync_copy(cat_hbm.at[pl.ds(base, bpw)], cidx)
        pltpu.sync_copy(ct_hbm, ctab)

        kv16 = jax.lax.iota(jnp.int32, 16)
        for n in range(nchunks):
            for j in range(_CHUNK // 16):
                vidx = cidx[pl.ds(n * _CHUNK + j * 16, 16)]
                fidx = jax.lax.shift_left(vidx, 3)
                kvec = kv16 + j * 16
                for col in range(ec):
                    cv = jnp.full((16,), col, jnp.int32)
                    val = plsc.load_gather(ctab, [fidx + cv])
                    plsc.store_scatter(csel, [kvec, cv], val)
            pltpu.sync_copy(
                csel, c_out.at[pl.ds(base + n * _CHUNK, _CHUNK)])

    return k(cat, ct_flat)


_BLK = 2048


def _h_body(u_ref, i_ref, c_ref, d_ref, dwt_ref, db_ref,
            w1u_ref, w1i_ref, w1c_ref, w1d_ref, b1_ref,
            h_ref, sum_ref, sq_ref):
    dd = jnp.maximum(
        jnp.dot(d_ref[...], dwt_ref[...], precision=_HIGH) + db_ref[...], 0.0)
    cc = c_ref[...][:, :w1c_ref.shape[0]]
    h = (jnp.dot(u_ref[...], w1u_ref[...], precision=_HIGH)
         + jnp.dot(i_ref[...], w1i_ref[...], precision=_HIGH)
         + jnp.dot(cc, w1c_ref[...], precision=_HIGH)
         + jnp.dot(dd, w1d_ref[...], precision=_HIGH)
         + b1_ref[...])
    h_ref[...] = h
    sum_ref[...] = jnp.sum(h, axis=0, keepdims=True)[None]
    sq_ref[...] = jnp.sum(h * h, axis=0, keepdims=True)[None]


def _norm_body(h_ref, sum_ref, sq_ref, g_ref, bb_ref,
               w2t_ref, b2_ref, wot_ref, bo_ref, o_ref, *, batch):
    mean = jnp.sum(sum_ref[...], axis=0) / batch
    var = jnp.sum(sq_ref[...], axis=0) / batch - mean * mean
    h = h_ref[...]
    hn = (h - mean) * jax.lax.rsqrt(var + 1e-5) * g_ref[...] + bb_ref[...]
    x = jnp.maximum(hn, 0.0)
    x = jnp.maximum(
        jnp.dot(x, w2t_ref[...], precision=_HIGH) + b2_ref[...], 0.0)
    o_ref[...] = jnp.dot(x, wot_ref[...], precision=_HIGH) + bo_ref[...]


def _tc_mlp(u, i, c, dense, dense_W, dense_b, fc1_W, fc1_b,
            bn_gamma, bn_beta, fc2_W, fc2_b, out_W, out_b):
    B = u.shape[0]
    eu = u.shape[1]
    ec = 8
    cw = c.shape[1]
    nb = B // _BLK
    w1t = fc1_W.T
    hdim = fc1_W.shape[0]

    def rows(bs):
        return pl.BlockSpec((_BLK, bs), lambda b: (b, 0))

    def full(shape):
        return pl.BlockSpec(shape, lambda b: (0,) * len(shape))

    h, sums, sqs = pl.pallas_call(
        _h_body,
        grid=(nb,),
        in_specs=[rows(eu), rows(eu), rows(cw), rows(2),
                  full((2, 8)), full((1, 8)),
                  full((eu, hdim)), full((eu, hdim)), full((ec, hdim)),
                  full((8, hdim)), full((1, hdim))],
        out_specs=[rows(hdim),
                   pl.BlockSpec((1, 1, hdim), lambda b: (b, 0, 0)),
                   pl.BlockSpec((1, 1, hdim), lambda b: (b, 0, 0))],
        out_shape=[jax.ShapeDtypeStruct((B, hdim), jnp.float32),
                   jax.ShapeDtypeStruct((nb, 1, hdim), jnp.float32),
                   jax.ShapeDtypeStruct((nb, 1, hdim), jnp.float32)],
    )(u, i, c, dense, dense_W.T, dense_b[None, :],
      w1t[:eu], w1t[eu:2 * eu], w1t[2 * eu:2 * eu + ec], w1t[2 * eu + ec:],
      fc1_b[None, :])

    return pl.pallas_call(
        functools.partial(_norm_body, batch=float(B)),
        grid=(nb,),
        in_specs=[rows(hdim), full((nb, 1, hdim)), full((nb, 1, hdim)),
                  full((1, hdim)), full((1, hdim)),
                  full((hdim, 32)), full((1, 32)), full((32, 1)),
                  full((1, 1))],
        out_specs=rows(1),
        out_shape=jax.ShapeDtypeStruct((B, 1), jnp.float32),
    )(h, sums, sqs, bn_gamma[None, :], bn_beta[None, :],
      fc2_W.T, fc2_b[None, :], out_W.T, out_b[None, :])


def kernel(user, item, cat, dense, user_table, item_table, cat_table,
           dense_W, dense_b, fc1_W, fc1_b, bn_gamma, bn_beta,
           fc2_W, fc2_b, out_W, out_b):
    u = jnp.take(user_table, user, axis=0)
    i = jnp.take(item_table, item, axis=0)
    c = _sc_gather_cat(cat.astype(jnp.int32), cat_table)
    return _tc_mlp(u, i, c, dense, dense_W, dense_b, fc1_W, fc1_b,
                   bn_gamma, bn_beta, fc2_W, fc2_b, out_W, out_b)
